# baseline (speedup 1.0000x reference)
import functools

import jax
import jax.numpy as jnp
from jax import lax
from jax.experimental import pallas as pl
from jax.experimental.pallas import tpu as pltpu
from jax.experimental.pallas import tpu_sc as plsc

_INFO = plsc.get_sparse_core_info()
_NC, _NS = _INFO.num_cores, _INFO.num_subcores
_NW = _NC * _NS

_CHUNK = 400
_DP = 128


def _pad_table(table, dp):
  V, D = table.shape
  blk = 10000
  assert V % blk == 0

  def body(t_ref, o_ref):
    o_ref[:, :D] = t_ref[...]

  return pl.pallas_call(
      body,
      grid=(V // blk,),
      in_specs=[pl.BlockSpec((blk, D), lambda i: (i, 0))],
      out_specs=pl.BlockSpec((blk, dp), lambda i: (i, 0)),
      out_shape=jax.ShapeDtypeStruct((V, dp), jnp.float32),
  )(table)


def _make_gather(V: int, B: int):
  assert B % (_NW * 2 * _CHUNK) == 0
  b_per_w = B // _NW
  n_chunks = b_per_w // _CHUNK
  n_rounds = n_chunks // 2
  mesh = plsc.VectorSubcoreMesh(core_axis_name="c", subcore_axis_name="s")

  @functools.partial(
      pl.kernel,
      mesh=mesh,
      out_type=jax.ShapeDtypeStruct((B, _DP), jnp.float32),
      scratch_types=[
          pltpu.VMEM((b_per_w,), jnp.int32),
          pltpu.VMEM((_CHUNK, _DP), jnp.float32),
          pltpu.VMEM((_CHUNK, _DP), jnp.float32),
          pltpu.SemaphoreType.DMA,
          pltpu.SemaphoreType.DMA,
          pltpu.SemaphoreType.DMA,
          pltpu.SemaphoreType.DMA,
      ],
  )
  def gather_kernel(table_hbm, idx_hbm, out_hbm, idx_all, rows0, rows1,
                    gs0, gs1, ss0, ss1):
    wid = lax.axis_index("s") * _NC + lax.axis_index("c")
    base = wid * b_per_w
    rows = (rows0, rows1)
    gsem = (gs0, gs1)
    ssem = (ss0, ss1)

    pltpu.sync_copy(idx_hbm.at[pl.ds(base, b_per_w)], idx_all)

    def gather_start(g, b):
      pltpu.async_copy(
          table_hbm.at[idx_all.at[pl.ds(g * _CHUNK, _CHUNK)]], rows[b], gsem[b]
      )

    def store_start(g, b):
      pltpu.async_copy(
          rows[b], out_hbm.at[pl.ds(base + g * _CHUNK, _CHUNK)], ssem[b]
      )

    def gather_wait(b):
      pltpu.make_async_copy(
          table_hbm.at[pl.ds(0, _CHUNK)], rows[b], gsem[b]
      ).wait()

    def store_wait(b):
      pltpu.make_async_copy(
          rows[b], out_hbm.at[pl.ds(base, _CHUNK)], ssem[b]
      ).wait()

    gather_start(0, 0)
    gather_start(1, 1)
    gather_wait(0)
    store_start(0, 0)

    def round_body(r, carry):
      g0 = 2 * r
      store_wait(0)
      gather_start(g0, 0)
      gather_wait(1)
      store_start(g0 - 1, 1)
      store_wait(1)
      gather_start(g0 + 1, 1)
      gather_wait(0)
      store_start(g0, 0)
      return carry

    lax.fori_loop(1, n_rounds, round_body, 0)

    gather_wait(1)
    store_start(n_chunks - 1, 1)
    store_wait(0)
    store_wait(1)

  return gather_kernel


def kernel(tokens, table):
  Bq, S = tokens.shape
  V, D = table.shape
  idx = tokens.reshape(-1).astype(jnp.int32)
  table_p = _pad_table(table, _DP)
  out = _make_gather(V, Bq * S)(table_p, idx)
  return out[:, :D].reshape(Bq, S, D)


# speedup vs baseline: 1.0033x; 1.0033x over previous
import functools

import jax
import jax.numpy as jnp
from jax import lax
from jax.experimental import pallas as pl
from jax.experimental.pallas import tpu as pltpu
from jax.experimental.pallas import tpu_sc as plsc

_INFO = plsc.get_sparse_core_info()
_NC, _NS = _INFO.num_cores, _INFO.num_subcores
_NW = _NC * _NS

_CHUNK = 200
_NBUF = 4
_DP = 128


def _pad_table(table, dp):
  V, D = table.shape
  blk = 10000
  assert V % blk == 0

  def body(t_ref, o_ref):
    o_ref[:, :D] = t_ref[...]

  return pl.pallas_call(
      body,
      grid=(V // blk,),
      in_specs=[pl.BlockSpec((blk, D), lambda i: (i, 0))],
      out_specs=pl.BlockSpec((blk, dp), lambda i: (i, 0)),
      out_shape=jax.ShapeDtypeStruct((V, dp), jnp.float32),
  )(table)


def _make_gather(V: int, B: int):
  assert B % (_NW * _NBUF * _CHUNK) == 0
  b_per_w = B // _NW
  n_chunks = b_per_w // _CHUNK
  n_rounds = n_chunks // _NBUF
  mesh = plsc.VectorSubcoreMesh(core_axis_name="c", subcore_axis_name="s")

  @functools.partial(
      pl.kernel,
      mesh=mesh,
      out_type=jax.ShapeDtypeStruct((B, _DP), jnp.float32),
      scratch_types=[
          pltpu.VMEM((b_per_w,), jnp.int32),
      ] + [pltpu.VMEM((_CHUNK, _DP), jnp.float32)] * _NBUF
        + [pltpu.SemaphoreType.DMA] * (2 * _NBUF),
  )
  def gather_kernel(table_hbm, idx_hbm, out_hbm, idx_all, *bufs):
    rows = bufs[:_NBUF]
    gsem = bufs[_NBUF:2 * _NBUF]
    ssem = bufs[2 * _NBUF:]
    wid = lax.axis_index("s") * _NC + lax.axis_index("c")
    base = wid * b_per_w

    pltpu.sync_copy(idx_hbm.at[pl.ds(base, b_per_w)], idx_all)

    def gather_start(g, b):
      pltpu.async_copy(
          table_hbm.at[idx_all.at[pl.ds(g * _CHUNK, _CHUNK)]], rows[b], gsem[b]
      )

    def store_start(g, b):
      pltpu.async_copy(
          rows[b], out_hbm.at[pl.ds(base + g * _CHUNK, _CHUNK)], ssem[b]
      )

    def gather_wait(b):
      pltpu.make_async_copy(
          table_hbm.at[pl.ds(0, _CHUNK)], rows[b], gsem[b]
      ).wait()

    def store_wait(b):
      pltpu.make_async_copy(
          rows[b], out_hbm.at[pl.ds(base, _CHUNK)], ssem[b]
      ).wait()

    gather_start(0, 0)
    gather_start(1, 1)
    for g in range(2, _NBUF):
      gather_start(g, g)
      gather_wait(g - 2)
      store_start(g - 2, g - 2)

    def round_body(r, carry):
      g0 = _NBUF * r
      for j in range(_NBUF):
        b = j
        store_wait(b)
        gather_start(g0 + j, b)
        b2 = (j - 2) % _NBUF
        gather_wait(b2)
        store_start(g0 + j - 2, b2)
      return carry

    lax.fori_loop(1, n_rounds, round_body, 0)

    for g in (n_chunks - 2, n_chunks - 1):
      b = g % _NBUF
      gather_wait(b)
      store_start(g, b)
    for b in range(_NBUF):
      store_wait(b)

  return gather_kernel


def kernel(tokens, table):
  Bq, S = tokens.shape
  V, D = table.shape
  idx = tokens.reshape(-1).astype(jnp.int32)
  table_p = _pad_table(table, _DP)
  out = _make_gather(V, Bq * S)(table_p, idx)
  return out[:, :D].reshape(Bq, S, D)
